# baseline (speedup 1.0000x reference)
import functools

import jax
import jax.numpy as jnp
from jax import lax
from jax.experimental import pallas as pl
from jax.experimental.pallas import tpu as pltpu
from jax.experimental.pallas import tpu_sc as plsc

_EMB_DIM = 32
_NUM_NODES = 1000000
_NUM_REL = 100
_E = 16384
_INFO = plsc.get_sparse_core_info()
_NC = _INFO.num_cores
_NS = _INFO.num_subcores
_NW = _NC * _NS
_EPS = _E // _NS
_NTC = 7812
_TAIL0 = _NTC * 128
_NTAIL = _NUM_NODES - _TAIL0
_TC0 = (0, 3906)
_TCHI = (3906, 7812)
_W = 128
_WN = _W * 128
_NWAVE = 32
_SB = 128


def _k1_body(h_idx, t_idx, r_idx, node3, rel, xout,
             sh0, sh1, vh, vt, vrr, vx, vbj, vbe, vaddr, vdst, vrtab,
             semf, semg):
    c = lax.axis_index("c")
    s = lax.axis_index("s")
    tc0 = jnp.where(c == 0, _TC0[0], _TC0[1])
    tchi = jnp.where(c == 0, _TCHI[0], _TCHI[1])
    ebase = s * _EPS
    iota = lax.iota(jnp.int32, 16)

    pltpu.sync_copy(h_idx.at[pl.ds(ebase, _EPS)], vh)
    pltpu.sync_copy(t_idx.at[pl.ds(ebase, _EPS)], vt)
    pltpu.sync_copy(r_idx.at[pl.ds(ebase, _EPS)], vrr)

    pltpu.sync_copy(rel, vrtab)

    def _zb(i, _):
        vbj[pl.ds(i * 16, 16)] = jnp.zeros((16,), jnp.int32)
        return _
    lax.fori_loop(0, 2 * _EPS // 16, _zb, 0)

    @pl.when(c == 0)
    def _seed_r():
        def _rinit(g, _):
            ev32 = (iota + g * 16) * _EMB_DIM
            rv = vrr[pl.ds(g * 16, 16)]
            for d in range(_EMB_DIM):
                dsp = jnp.full((16,), d, jnp.int32)
                val = plsc.load_gather(vrtab, [rv, dsp])
                plsc.store_scatter(vx, [ev32 + d], val)
            return _
        lax.fori_loop(0, _EPS // 16, _rinit, 0)

    @pl.when(c != 0)
    def _seed_zero():
        def _zx(i, _):
            vx[pl.ds(i * 16, 16)] = jnp.zeros((16,), jnp.float32)
            return _
        lax.fori_loop(0, _EPS * _EMB_DIM // 16, _zx, 0)

    def _off(w):
        off = jnp.minimum(tc0 + w * _W, tchi - _W)
        return pl.multiple_of(off * 128, 128)

    def _fire(w, buf):
        off = _off(w)
        for k in range(2):
            d = s * 2 + k
            pltpu.async_copy(
                node3.at[d // 8, d % 8].at[pl.ds(off, _WN)],
                buf.at[pl.ds(d * _WN, _WN)], semf)

    def _drain(w, buf):
        off = _off(w)
        for k in range(2):
            d = s * 2 + k
            pltpu.make_async_copy(
                node3.at[d // 8, d % 8].at[pl.ds(off, _WN)],
                buf.at[pl.ds(d * _WN, _WN)], semf).wait()

    def _process(w, buf):
        win = _off(w)
        lo = jnp.minimum(tc0 + w * _W, tchi) * 128
        hi = jnp.minimum(tc0 + (w + 1) * _W, tchi) * 128

        for vidx, neg in ((vh, False), (vt, True)):
            def _filt(g, cnt):
                iv = vidx[pl.ds(g * 16, 16)]
                m = (iv >= lo) & (iv < hi)
                mi = jnp.where(m, 1, 0).astype(jnp.int32)
                cum = plsc.cumsum(mi)
                pos = jnp.broadcast_to(cnt, (16,)) + cum - mi
                plsc.store_scatter(vbj, [pos], iv - win, mask=m)
                plsc.store_scatter(vbe, [pos], iota + g * 16, mask=m)
                return cnt + jnp.sum(mi)
            cnt = lax.fori_loop(0, _EPS // 16, _filt, jnp.int32(0))

            nb = (cnt + (_SB - 1)) // _SB

            def _sub(k, _):
                kb = k * _SB
                def _bld(g2, __):
                    q0 = g2 * 16
                    jv = vbj[pl.ds(kb + q0, 16)]
                    qpos = (iota + q0) * _EMB_DIM
                    for d in range(_EMB_DIM):
                        plsc.store_scatter(vaddr, [qpos + d], jv + d * _WN)
                    return __
                lax.fori_loop(0, _SB // 16, _bld, 0)
                pltpu.async_copy(buf.at[vaddr], vdst, semg).wait()

                def _add(g2, __):
                    q0 = g2 * 16
                    live = (iota + (kb + q0)) < cnt
                    ev32 = vbe[pl.ds(kb + q0, 16)] * _EMB_DIM
                    qpos = (iota + q0) * _EMB_DIM
                    for d in range(_EMB_DIM):
                        val = plsc.load_gather(vdst, [qpos + d])
                        if neg:
                            val = -val
                        plsc.addupdate_scatter(vx, [ev32 + d], val,
                                               mask=live)
                    return __
                lax.fori_loop(0, _SB // 16, _add, 0)
                return _
            lax.fori_loop(0, nb, _sub, 0)

    _fire(0, sh0)

    def _pair(p, carry):
        w = p * 2
        _fire(w + 1, sh1)
        _drain(w, sh0)
        plsc.subcore_barrier()
        _process(w, sh0)
        plsc.subcore_barrier()

        @pl.when(p + 1 < _NWAVE // 2)
        def _fire_ahead():
            _fire(w + 2, sh0)
        _drain(w + 1, sh1)
        plsc.subcore_barrier()
        _process(w + 1, sh1)
        plsc.subcore_barrier()
        return carry
    lax.fori_loop(0, _NWAVE // 2, _pair, 0)

    pltpu.sync_copy(vx, xout.at[c].at[pl.ds(ebase * _EMB_DIM, _EPS * _EMB_DIM)])


_k1 = functools.partial(
    pl.kernel,
    out_type=jax.ShapeDtypeStruct((_NC, _E * _EMB_DIM), jnp.float32),
    mesh=plsc.VectorSubcoreMesh(core_axis_name="c", subcore_axis_name="s"),
    compiler_params=pltpu.CompilerParams(needs_layout_passes=False),
    scratch_types=[
        pltpu.VMEM_SHARED((_EMB_DIM * _WN,), jnp.float32),
        pltpu.VMEM_SHARED((_EMB_DIM * _WN,), jnp.float32),
        pltpu.VMEM((_EPS,), jnp.int32),
        pltpu.VMEM((_EPS,), jnp.int32),
        pltpu.VMEM((_EPS,), jnp.int32),
        pltpu.VMEM((_EPS * _EMB_DIM,), jnp.float32),
        pltpu.VMEM((2 * _EPS,), jnp.int32),
        pltpu.VMEM((2 * _EPS,), jnp.int32),
        pltpu.VMEM((_SB * _EMB_DIM,), jnp.int32),
        pltpu.VMEM((_SB * _EMB_DIM,), jnp.float32),
        pltpu.VMEM((_NUM_REL, _EMB_DIM), jnp.float32),
        pltpu.SemaphoreType.DMA,
        pltpu.SemaphoreType.DMA,
    ],
)(_k1_body)


def _k2_body(x_ref, h_ref, t_ref, tail_ref, o_ref):
    x = x_ref[0] + x_ref[1]
    ids = lax.broadcasted_iota(jnp.int32, (_E, _NTAIL), 1) + _TAIL0
    h = h_ref[...].reshape(_E, 1)
    t = t_ref[...].reshape(_E, 1)
    oh = (h == ids).astype(jnp.float32) - (t == ids).astype(jnp.float32)
    tail = tail_ref[:, 0:_NTAIL]
    x = x + lax.dot_general(oh, tail, (((1,), (1,)), ((), ())))
    o_ref[0, 0] = jnp.sum(x * x)


def _k2(xout, h_idx, t_idx, node2):
    return pl.pallas_call(
        _k2_body,
        grid=(1,),
        out_shape=jax.ShapeDtypeStruct((1, 1), jnp.float32),
        in_specs=[
            pl.BlockSpec(xout.shape, lambda i: (0, 0, 0)),
            pl.BlockSpec(h_idx.shape, lambda i: (0,)),
            pl.BlockSpec(t_idx.shape, lambda i: (0,)),
            pl.BlockSpec((_EMB_DIM, 128), lambda i: (0, _TAIL0 // 128)),
        ],
        out_specs=pl.BlockSpec((1, 1), lambda i: (0, 0),
                               memory_space=pltpu.SMEM),
    )(xout, h_idx, t_idx, node2)


@jax.jit
def kernel(edge_index_t, edge_attr, node_emb_weight, r_emb_weight):
    h_idx = edge_index_t[0]
    t_idx = edge_index_t[1]
    r_idx = edge_attr[:, 0]
    node2 = node_emb_weight.T
    node3 = node2.reshape(_EMB_DIM // 8, 8, _NUM_NODES)
    xout = _k1(h_idx, t_idx, r_idx, node3, r_emb_weight)
    xout = xout.reshape(_NC, _E, _EMB_DIM)
    loss_sum = _k2(xout, h_idx, t_idx, node2)
    return loss_sum[0, 0] * (1.0 / (_E * _EMB_DIM))


# speedup vs baseline: 4.4991x; 4.4991x over previous
import functools

import jax
import jax.numpy as jnp
from jax import lax
from jax.experimental import pallas as pl
from jax.experimental.pallas import tpu as pltpu
from jax.experimental.pallas import tpu_sc as plsc

_EMB_DIM = 32
_E = 16384
_INFO = plsc.get_sparse_core_info()
_NC = _INFO.num_cores
_NS = _INFO.num_subcores
_NW = _NC * _NS
_EPW = _E // _NW
_CHUNK = 128
_NCHUNK = _EPW // _CHUNK


def _sc_body(h_idx_hbm, t_idx_hbm, r_idx_hbm, node_hbm, rel_hbm, out_hbm,
             vh_idx, vt_idx, vr_idx, vh, vt, vr, vacc, sem):
    c = lax.axis_index("c")
    s = lax.axis_index("s")
    wid = s * _NC + c
    base = wid * _NCHUNK

    pltpu.sync_copy(h_idx_hbm.at[pl.ds(base, _NCHUNK)], vh_idx)
    pltpu.sync_copy(t_idx_hbm.at[pl.ds(base, _NCHUNK)], vt_idx)
    pltpu.sync_copy(r_idx_hbm.at[pl.ds(base, _NCHUNK)], vr_idx)

    copies = []
    for j in range(_NCHUNK):
        dst = pl.ds(j * _CHUNK, _CHUNK)
        copies.append(pltpu.async_copy(node_hbm.at[vh_idx.at[j]], vh.at[dst], sem))
        copies.append(pltpu.async_copy(node_hbm.at[vt_idx.at[j]], vt.at[dst], sem))
        copies.append(pltpu.async_copy(rel_hbm.at[vr_idx.at[j]], vr.at[dst], sem))
    for cp in copies:
        cp.wait()

    def step(i, accs):
        a0, a1 = accs
        d0 = vh[i, pl.ds(0, 16)] + vr[i, pl.ds(0, 16)] - vt[i, pl.ds(0, 16)]
        d1 = vh[i, pl.ds(16, 16)] + vr[i, pl.ds(16, 16)] - vt[i, pl.ds(16, 16)]
        return (a0 + d0 * d0, a1 + d1 * d1)

    zero = jnp.zeros((16,), jnp.float32)
    a0, a1 = lax.fori_loop(0, _EPW, step, (zero, zero))
    vacc[...] = a0 + a1
    pltpu.sync_copy(vacc, out_hbm.at[wid])


_sc_call = functools.partial(
    pl.kernel,
    out_type=jax.ShapeDtypeStruct((_NW, 16), jnp.float32),
    mesh=plsc.VectorSubcoreMesh(core_axis_name="c", subcore_axis_name="s"),
    compiler_params=pltpu.CompilerParams(use_tc_tiling_on_sc=False),
    scratch_types=[
        pltpu.VMEM((_NCHUNK, _CHUNK), jnp.int32),
        pltpu.VMEM((_NCHUNK, _CHUNK), jnp.int32),
        pltpu.VMEM((_NCHUNK, _CHUNK), jnp.int32),
        pltpu.VMEM((_EPW, _EMB_DIM), jnp.float32),
        pltpu.VMEM((_EPW, _EMB_DIM), jnp.float32),
        pltpu.VMEM((_EPW, _EMB_DIM), jnp.float32),
        pltpu.VMEM((16,), jnp.float32),
        pltpu.SemaphoreType.DMA,
    ],
)(_sc_body)


@jax.jit
def kernel(edge_index_t, edge_attr, node_emb_weight, r_emb_weight):
    h_idx = edge_index_t[0].reshape(_NW * _NCHUNK, _CHUNK)
    t_idx = edge_index_t[1].reshape(_NW * _NCHUNK, _CHUNK)
    r_idx = edge_attr[:, 0].reshape(_NW * _NCHUNK, _CHUNK)
    partials = _sc_call(h_idx, t_idx, r_idx, node_emb_weight, r_emb_weight)
    return jnp.sum(partials) * (1.0 / (_E * _EMB_DIM))
